# Initial kernel scaffold; baseline (speedup 1.0000x reference)
import jax
import jax.numpy as jnp
from jax.experimental import pallas as pl


def kernel(x, Pi, high_centroids, low_centroids, high_indices, low_indices):
    raise NotImplementedError("write your pallas kernel here")


# speedup vs baseline: 7.0078x; 7.0078x over previous
import functools

import jax
import jax.numpy as jnp
from jax.experimental import pallas as pl

_D = 128
_K = 16
_BLK = 2048


def _body(x_ref, pit_ref, pi_ref, mids_ref, deltas_ref, o_ref):
    y = jnp.dot(x_ref[...], pit_ref[...], preferred_element_type=jnp.float32)
    yq = jnp.zeros_like(y)
    for k in range(_K):
        yq = yq + deltas_ref[k : k + 1, :] * (y > mids_ref[k : k + 1, :]).astype(
            jnp.float32
        )
    o_ref[...] = jnp.dot(yq, pi_ref[...], preferred_element_type=jnp.float32)


@functools.partial(jax.jit, static_argnames=())
def kernel(x, Pi, high_centroids, low_centroids, high_indices, low_indices):
    B = x.shape[0]
    is_high = jnp.zeros((_D,), jnp.bool_).at[high_indices].set(True)
    low_rep = jnp.repeat(low_centroids, _K // low_centroids.shape[0])
    tbl = jnp.where(is_high[:, None], high_centroids[None, :], low_rep[None, :])
    mids = jnp.concatenate(
        [jnp.full((_D, 1), -3.4e38, jnp.float32), 0.5 * (tbl[:, 1:] + tbl[:, :-1])],
        axis=1,
    ).T
    deltas = jnp.concatenate([tbl[:, :1], tbl[:, 1:] - tbl[:, :-1]], axis=1).T

    grid = (B // _BLK,)
    return pl.pallas_call(
        _body,
        grid=grid,
        in_specs=[
            pl.BlockSpec((_BLK, _D), lambda i: (i, 0)),
            pl.BlockSpec((_D, _D), lambda i: (0, 0)),
            pl.BlockSpec((_D, _D), lambda i: (0, 0)),
            pl.BlockSpec((_K, _D), lambda i: (0, 0)),
            pl.BlockSpec((_K, _D), lambda i: (0, 0)),
        ],
        out_specs=pl.BlockSpec((_BLK, _D), lambda i: (i, 0)),
        out_shape=jax.ShapeDtypeStruct((B, _D), jnp.float32),
    )(x, Pi.T, Pi, mids, deltas)


# speedup vs baseline: 9.8348x; 1.4034x over previous
import functools

import jax
import jax.numpy as jnp
from jax.experimental import pallas as pl

_D = 128
_K = 16
_BLK = 2048


def _body(x_ref, pit_ref, pi_ref, mids_ref, tbl_ref, o_ref):
    y = jnp.dot(x_ref[...], pit_ref[...], preferred_element_type=jnp.float32)

    def m(k):
        return mids_ref[k : k + 1, :]

    def t(k):
        return tbl_ref[k : k + 1, :]

    w = jnp.where
    b3 = y > m(8)
    b2 = y > w(b3, m(12), m(4))
    b1 = y > w(b3, w(b2, m(14), m(10)), w(b2, m(6), m(2)))
    b0 = y > w(
        b3,
        w(b2, w(b1, m(15), m(13)), w(b1, m(11), m(9))),
        w(b2, w(b1, m(7), m(5)), w(b1, m(3), m(1))),
    )
    yq = w(
        b3,
        w(
            b2,
            w(b1, w(b0, t(15), t(14)), w(b0, t(13), t(12))),
            w(b1, w(b0, t(11), t(10)), w(b0, t(9), t(8))),
        ),
        w(
            b2,
            w(b1, w(b0, t(7), t(6)), w(b0, t(5), t(4))),
            w(b1, w(b0, t(3), t(2)), w(b0, t(1), t(0))),
        ),
    )
    o_ref[...] = jnp.dot(yq, pi_ref[...], preferred_element_type=jnp.float32)


@functools.partial(jax.jit, static_argnames=())
def kernel(x, Pi, high_centroids, low_centroids, high_indices, low_indices):
    B = x.shape[0]
    is_high = jnp.zeros((_D,), jnp.bool_).at[high_indices].set(True)
    low_rep = jnp.repeat(low_centroids, _K // low_centroids.shape[0])
    tbl = jnp.where(is_high[:, None], high_centroids[None, :], low_rep[None, :])
    mids = jnp.concatenate(
        [jnp.full((_D, 1), -3.4e38, jnp.float32), 0.5 * (tbl[:, 1:] + tbl[:, :-1])],
        axis=1,
    ).T
    tbl_t = tbl.T

    grid = (B // _BLK,)
    return pl.pallas_call(
        _body,
        grid=grid,
        in_specs=[
            pl.BlockSpec((_BLK, _D), lambda i: (i, 0)),
            pl.BlockSpec((_D, _D), lambda i: (0, 0)),
            pl.BlockSpec((_D, _D), lambda i: (0, 0)),
            pl.BlockSpec((_K, _D), lambda i: (0, 0)),
            pl.BlockSpec((_K, _D), lambda i: (0, 0)),
        ],
        out_specs=pl.BlockSpec((_BLK, _D), lambda i: (i, 0)),
        out_shape=jax.ShapeDtypeStruct((B, _D), jnp.float32),
    )(x, Pi.T, Pi, mids, tbl_t)


# speedup vs baseline: 11.9468x; 1.2147x over previous
import functools

import jax
import jax.numpy as jnp
from jax.experimental import pallas as pl

_D = 128
_K = 16
_BLK = 4096


def _body(x_ref, pit_ref, pi_ref, mids_ref, tbl_ref, o_ref):
    y = jnp.dot(x_ref[...], pit_ref[...], preferred_element_type=jnp.float32)

    def m(k):
        return mids_ref[k : k + 1, :]

    def t(k):
        return tbl_ref[k : k + 1, :]

    w = jnp.where
    b3 = y > m(8)
    b2 = y > w(b3, m(12), m(4))
    b1 = y > w(b3, w(b2, m(14), m(10)), w(b2, m(6), m(2)))
    b0 = y > w(
        b3,
        w(b2, w(b1, m(15), m(13)), w(b1, m(11), m(9))),
        w(b2, w(b1, m(7), m(5)), w(b1, m(3), m(1))),
    )
    yq = w(
        b3,
        w(
            b2,
            w(b1, w(b0, t(15), t(14)), w(b0, t(13), t(12))),
            w(b1, w(b0, t(11), t(10)), w(b0, t(9), t(8))),
        ),
        w(
            b2,
            w(b1, w(b0, t(7), t(6)), w(b0, t(5), t(4))),
            w(b1, w(b0, t(3), t(2)), w(b0, t(1), t(0))),
        ),
    )
    o_ref[...] = jnp.dot(yq, pi_ref[...], preferred_element_type=jnp.float32)


@functools.partial(jax.jit, static_argnames=())
def kernel(x, Pi, high_centroids, low_centroids, high_indices, low_indices):
    B = x.shape[0]
    is_high = jnp.zeros((_D,), jnp.bool_).at[high_indices].set(True)
    low_rep = jnp.repeat(low_centroids, _K // low_centroids.shape[0])
    tbl = jnp.where(is_high[:, None], high_centroids[None, :], low_rep[None, :])
    mids = jnp.concatenate(
        [jnp.full((_D, 1), -3.4e38, jnp.float32), 0.5 * (tbl[:, 1:] + tbl[:, :-1])],
        axis=1,
    ).T
    tbl_t = tbl.T

    grid = (B // _BLK,)
    return pl.pallas_call(
        _body,
        grid=grid,
        in_specs=[
            pl.BlockSpec((_BLK, _D), lambda i: (i, 0)),
            pl.BlockSpec((_D, _D), lambda i: (0, 0)),
            pl.BlockSpec((_D, _D), lambda i: (0, 0)),
            pl.BlockSpec((_K, _D), lambda i: (0, 0)),
            pl.BlockSpec((_K, _D), lambda i: (0, 0)),
        ],
        out_specs=pl.BlockSpec((_BLK, _D), lambda i: (i, 0)),
        out_shape=jax.ShapeDtypeStruct((B, _D), jnp.float32),
    )(x, Pi.T, Pi, mids, tbl_t)


# speedup vs baseline: 12.6062x; 1.0552x over previous
import functools

import jax
import jax.numpy as jnp
from jax.experimental import pallas as pl

_D = 128
_K = 16
_BLK = 8192


def _body(x_ref, pit_ref, pi_ref, mids_ref, tbl_ref, o_ref):
    y = jnp.dot(x_ref[...], pit_ref[...], preferred_element_type=jnp.float32)

    def m(k):
        return mids_ref[k : k + 1, :]

    def t(k):
        return tbl_ref[k : k + 1, :]

    w = jnp.where
    b3 = y > m(8)
    b2 = y > w(b3, m(12), m(4))
    b1 = y > w(b3, w(b2, m(14), m(10)), w(b2, m(6), m(2)))
    b0 = y > w(
        b3,
        w(b2, w(b1, m(15), m(13)), w(b1, m(11), m(9))),
        w(b2, w(b1, m(7), m(5)), w(b1, m(3), m(1))),
    )
    yq = w(
        b3,
        w(
            b2,
            w(b1, w(b0, t(15), t(14)), w(b0, t(13), t(12))),
            w(b1, w(b0, t(11), t(10)), w(b0, t(9), t(8))),
        ),
        w(
            b2,
            w(b1, w(b0, t(7), t(6)), w(b0, t(5), t(4))),
            w(b1, w(b0, t(3), t(2)), w(b0, t(1), t(0))),
        ),
    )
    o_ref[...] = jnp.dot(yq, pi_ref[...], preferred_element_type=jnp.float32)


@functools.partial(jax.jit, static_argnames=())
def kernel(x, Pi, high_centroids, low_centroids, high_indices, low_indices):
    B = x.shape[0]
    is_high = jnp.zeros((_D,), jnp.bool_).at[high_indices].set(True)
    low_rep = jnp.repeat(low_centroids, _K // low_centroids.shape[0])
    tbl = jnp.where(is_high[:, None], high_centroids[None, :], low_rep[None, :])
    mids = jnp.concatenate(
        [jnp.full((_D, 1), -3.4e38, jnp.float32), 0.5 * (tbl[:, 1:] + tbl[:, :-1])],
        axis=1,
    ).T
    tbl_t = tbl.T

    grid = (B // _BLK,)
    return pl.pallas_call(
        _body,
        grid=grid,
        in_specs=[
            pl.BlockSpec((_BLK, _D), lambda i: (i, 0)),
            pl.BlockSpec((_D, _D), lambda i: (0, 0)),
            pl.BlockSpec((_D, _D), lambda i: (0, 0)),
            pl.BlockSpec((_K, _D), lambda i: (0, 0)),
            pl.BlockSpec((_K, _D), lambda i: (0, 0)),
        ],
        out_specs=pl.BlockSpec((_BLK, _D), lambda i: (i, 0)),
        out_shape=jax.ShapeDtypeStruct((B, _D), jnp.float32),
    )(x, Pi.T, Pi, mids, tbl_t)
